# Initial kernel scaffold; baseline (speedup 1.0000x reference)
import jax
import jax.numpy as jnp
from jax.experimental import pallas as pl


def kernel(x, edge_index, W0_0, b0_0, W0_1, b0_1, W0_2, b0_2, bn_g, bn_b, W1_0, b1_0, W1_1, b1_1, W1_2, b1_2, Wf, bf):
    raise NotImplementedError("write your pallas kernel here")


# speedup vs baseline: 6.6291x; 6.6291x over previous
import functools

import jax
import jax.numpy as jnp
from jax import lax
from jax.experimental import pallas as pl
from jax.experimental.pallas import tpu as pltpu
from jax.experimental.pallas import tpu_sc as plsc

N = 10000
D = 128
NP = 10112
E = 320000
NC, NS, L = 2, 16, 16
NW = NC * NS
EPT = 10112
EPAD = EPT * NW
K = 128
CHUNKS = EPT // K
STRIPE = NP // NS

@functools.cache
def _mesh():
    return plsc.VectorSubcoreMesh(
        core_axis_name="c", subcore_axis_name="s", num_cores=NC, num_subcores=NS)


def _deg_body(colp_hbm, out_hbm, colv, ones_v, zbuf, acc, sem):
    cid = lax.axis_index("c")
    sid = lax.axis_index("s")
    wid = cid * NS + sid
    for i in range(STRIPE // L + 1):
        zbuf[pl.ds(i * L, L)] = jnp.zeros((L,), jnp.float32)
    for i in range(K // L):
        ones_v[pl.ds(i * L, L)] = jnp.full((L,), 1.0, jnp.float32)
    pltpu.sync_copy(zbuf.at[pl.ds(0, STRIPE)], acc.at[pl.ds(sid * STRIPE, STRIPE)])
    plsc.subcore_barrier()

    def body(i, carry):
        off = pl.multiple_of(wid * EPT + i * K, K)
        pltpu.sync_copy(colp_hbm.at[pl.ds(off, K)], colv)
        pltpu.sync_copy(ones_v, acc.at[colv], add=True)
        return carry

    lax.fori_loop(0, CHUNKS, body, 0)
    plsc.subcore_barrier()
    off = pl.multiple_of(cid * NP + sid * STRIPE, 8)
    pltpu.sync_copy(acc.at[pl.ds(sid * STRIPE, STRIPE)], zbuf.at[pl.ds(0, STRIPE)])
    pltpu.sync_copy(zbuf.at[pl.ds(0, STRIPE)], out_hbm.at[pl.ds(off, STRIPE)])


@functools.cache
def _sc_deg_kernel():
    return pl.kernel(
        _deg_body,
        out_type=jax.ShapeDtypeStruct((NC * NP,), jnp.float32),
        mesh=_mesh(),
        scratch_types=[
            pltpu.VMEM((K,), jnp.int32),
            pltpu.VMEM((K,), jnp.float32),
            pltpu.VMEM((STRIPE // L * L + L,), jnp.float32),
            pltpu.VMEM_SHARED((NP,), jnp.float32),
            pltpu.SemaphoreType.DMA,
        ],
    )


def _sc_deg(colp):
    return _sc_deg_kernel()(colp).reshape(NC, NP)


def _prop_body(u_hbm, colp_hbm, rowp_hbm, out_hbm, colv, rowv, rows_v, acc, sem):
    cid = lax.axis_index("c")
    sid = lax.axis_index("s")
    wid = cid * NS + sid
    for o, sz in ((0, K), (K, K), (2 * K, K), (3 * K, K), (4 * K, STRIPE - 4 * K)):
        pltpu.sync_copy(u_hbm.at[pl.ds(sid * STRIPE + o, sz)], rows_v.at[pl.ds(0, sz)])
        pltpu.sync_copy(rows_v.at[pl.ds(0, sz)], acc.at[pl.ds(sid * STRIPE + o, sz)])
    plsc.subcore_barrier()

    def body(i, carry):
        off = pl.multiple_of(wid * EPT + i * K, K)
        pltpu.sync_copy(colp_hbm.at[pl.ds(off, K)], colv)
        pltpu.sync_copy(rowp_hbm.at[pl.ds(off, K)], rowv)
        pltpu.async_copy(u_hbm.at[colv], rows_v, sem).wait()
        pltpu.sync_copy(rows_v, acc.at[rowv], add=True)
        return carry

    lax.fori_loop(0, CHUNKS, body, 0)
    plsc.subcore_barrier()
    for o, sz in ((0, K), (K, K), (2 * K, K), (3 * K, K), (4 * K, STRIPE - 4 * K)):
        pltpu.sync_copy(acc.at[pl.ds(sid * STRIPE + o, sz)], rows_v.at[pl.ds(0, sz)])
        pltpu.sync_copy(rows_v.at[pl.ds(0, sz)],
                        out_hbm.at[cid, pl.ds(sid * STRIPE + o, sz)])


@functools.cache
def _sc_prop_kernel():
    return pl.kernel(
        _prop_body,
        out_type=jax.ShapeDtypeStruct((NC, NP, D), jnp.float32),
        mesh=_mesh(),
        scratch_types=[
            pltpu.VMEM((K,), jnp.int32),
            pltpu.VMEM((K,), jnp.int32),
            pltpu.VMEM((K, D), jnp.float32),
            pltpu.VMEM_SHARED((NP, D), jnp.float32),
            pltpu.SemaphoreType.DMA,
        ],
    )


def _sc_prop(u, colp, rowp):
    return _sc_prop_kernel()(u, colp, rowp)


def kernel(x, edge_index, W0_0, b0_0, W0_1, b0_1, W0_2, b0_2, bn_g, bn_b,
           W1_0, b1_0, W1_1, b1_1, W1_2, b1_2, Wf, bf):
    row = edge_index[0]
    col = edge_index[1]
    pad_idx = jnp.full((EPAD - E,), N, jnp.int32)
    rowp = jnp.concatenate([row, pad_idx])
    colp = jnp.concatenate([col, pad_idx])

    degp = _sc_deg(colp)
    deg = degp[0] + degp[1] + 1.0
    dis = lax.rsqrt(deg)[:, None]

    xp = jnp.pad(x, ((0, NP - N), (0, 0)))

    def prop(tp):
        u = dis * tp
        p = _sc_prop(u, colp, rowp)
        return dis * (p[0] + p[1] - u)

    def mixhop(hp, Ws, bs):
        outs = []
        for j, (W, b) in enumerate(zip(Ws, bs)):
            hj = hp @ W.T + b
            for _ in range(j):
                hj = prop(hj)
            outs.append(hj)
        return jnp.concatenate(outs, axis=1)

    h = mixhop(xp, [W0_0, W0_1, W0_2], [b0_0, b0_1, b0_2])
    hn = h[:N]
    mean = hn.mean(axis=0)
    var = hn.var(axis=0)
    hn = (hn - mean) / jnp.sqrt(var + 1e-5) * bn_g + bn_b
    hn = jax.nn.relu(hn)
    hp = jnp.pad(hn, ((0, NP - N), (0, 0)))
    h2 = mixhop(hp, [W1_0, W1_1, W1_2], [b1_0, b1_1, b1_2])
    return (h2 @ Wf.T + bf)[:N]
